# baseline (speedup 1.0000x reference)
import functools

import jax
import jax.numpy as jnp
from jax import lax
from jax.experimental import pallas as pl
from jax.experimental.pallas import tpu as pltpu
from jax.experimental.pallas import tpu_sc as plsc

NUM_CORES = 2
NUM_SUBCORES = 16
LANES = 16
NUM_WORKERS = NUM_CORES * NUM_SUBCORES
BLK = 128
WAVE = 16
DEPTH = 5
NBKT = 16


def _rsqrt_newton(x):
    i = lax.bitcast_convert_type(x, jnp.int32)
    i = jnp.int32(0x5F3759DF) - lax.shift_right_logical(i, 1)
    y = lax.bitcast_convert_type(i, jnp.float32)
    for _ in range(2):
        y = y * (1.5 - 0.5 * x * y * y)
    return y


def _make_sc_call(batch, feat_dim, num_classes):
    nblocks = (num_classes + BLK - 1) // BLK
    npb = -(-((nblocks + NUM_WORKERS - 1) // NUM_WORKERS) // DEPTH) * DEPTH
    nslices = feat_dim // LANES

    mesh = plsc.VectorSubcoreMesh(core_axis_name="c", subcore_axis_name="s")

    @functools.partial(
        pl.kernel,
        mesh=mesh,
        compiler_params=pltpu.CompilerParams(needs_layout_passes=False),
        out_type=jax.ShapeDtypeStruct((NUM_WORKERS, LANES), jnp.float32),
        scratch_types=[
            pltpu.VMEM((batch + LANES,), jnp.int32),
            pltpu.VMEM((batch + LANES,), jnp.int32),
            pltpu.VMEM((batch + LANES,), jnp.int32),
            pltpu.VMEM((batch + LANES,), jnp.int32),
            pltpu.VMEM((DEPTH, feat_dim, BLK), jnp.float32),
            pltpu.VMEM((DEPTH, WAVE * feat_dim), jnp.float32),
            pltpu.VMEM((LANES,), jnp.float32),
            pltpu.SMEM((256,), jnp.int32),
        ] + [pltpu.SemaphoreType.DMA] * DEPTH,
    )
    def sc_kernel(feat_hbm, lab_hbm, cent_hbm, out_hbm, lab_v, mylab_v,
                  myidx_v, bidx_v, slab_v, fr_v, out_v, off_s,
                  *sems):
        arena_v = mylab_v
        wid = lax.axis_index("s") * NUM_CORES + lax.axis_index("c")
        pltpu.sync_copy(lab_hbm, lab_v.at[pl.ds(0, batch)])

        frange = lax.iota(jnp.int32, LANES)
        zeros = jnp.zeros((LANES,), jnp.float32)

        def popc(mask):
            return plsc.all_reduce_population_count(mask)[0]

        def scan_body(v, cnt):
            lv = lab_v[pl.ds(v * LANES, LANES)]
            m = jnp.bitwise_and(lax.shift_right_logical(lv, 7),
                                NUM_WORKERS - 1) == wid
            plsc.store_compressed(mylab_v.at[pl.ds(cnt, LANES)], lv, mask=m)
            plsc.store_compressed(myidx_v.at[pl.ds(cnt, LANES)],
                                  v * LANES + frange, mask=m)
            return cnt + popc(m)

        cnt = lax.fori_loop(0, batch // LANES, scan_body, jnp.int32(0))
        nv = lax.shift_right_logical(cnt + LANES - 1, 4)

        def bcount(v, cs):
            lv = mylab_v[pl.ds(v * LANES, LANES)]
            valid = (v * LANES + frange) < cnt
            t = lax.shift_right_logical(lv, 16)
            return tuple(
                cs[j] + popc(jnp.logical_and(t == j, valid))
                for j in range(NBKT)
            )

        bcnt = lax.fori_loop(0, nv, bcount, (jnp.int32(0),) * NBKT)
        boff = [jnp.int32(0)]
        for j in range(NBKT):
            boff.append(boff[j] + bcnt[j])

        def bplace(v, cur):
            lv = mylab_v[pl.ds(v * LANES, LANES)]
            iv = myidx_v[pl.ds(v * LANES, LANES)]
            valid = (v * LANES + frange) < cnt
            t = lax.shift_right_logical(lv, 16)
            out = []
            for j in range(NBKT):
                m = jnp.logical_and(t == j, valid)
                plsc.store_compressed(lab_v.at[pl.ds(cur[j], LANES)],
                                      lv, mask=m)
                plsc.store_compressed(bidx_v.at[pl.ds(cur[j], LANES)],
                                      iv, mask=m)
                out.append(cur[j] + popc(m))
            return tuple(out)

        lax.fori_loop(0, nv, bplace, tuple(boff[:NBKT]))

        for j in range(NBKT):
            lo = boff[j]
            hi = boff[j + 1]
            nvj = lax.shift_right_logical(hi - lo + LANES - 1, 4)

            def ccount(u, cs, lo=lo, hi=hi):
                p = lo + u * LANES
                lv = lab_v[pl.ds(p, LANES)]
                valid = (p + frange) < hi
                k = jnp.bitwise_and(lax.shift_right_logical(lv, 12),
                                    NBKT - 1)
                return tuple(
                    cs[q] + popc(jnp.logical_and(k == q, valid))
                    for q in range(NBKT)
                )

            ccnt = lax.fori_loop(0, nvj, ccount, (jnp.int32(0),) * NBKT)
            base = lo
            curs = []
            for q in range(NBKT):
                off_s[j * NBKT + q] = base
                curs.append(base)
                base = base + ccnt[q]

            def cplace(u, cur, lo=lo, hi=hi):
                p = lo + u * LANES
                lv = lab_v[pl.ds(p, LANES)]
                iv = bidx_v[pl.ds(p, LANES)]
                valid = (p + frange) < hi
                k = jnp.bitwise_and(lax.shift_right_logical(lv, 12),
                                    NBKT - 1)
                pk = iv * BLK + jnp.bitwise_and(lv, BLK - 1)
                out = []
                for q in range(NBKT):
                    m = jnp.logical_and(k == q, valid)
                    plsc.store_compressed(arena_v.at[pl.ds(cur[q], LANES)],
                                          pk, mask=m)
                    out.append(cur[q] + popc(m))
                return tuple(out)

            lax.fori_loop(0, nvj, cplace, tuple(curs))

        def stage(b_loc, s):
            lb = b_loc * NUM_WORKERS + wid
            st = off_s[b_loc]
            eb = off_s[b_loc + 1] - st

            @pl.when(eb > 0)
            def _():
                start = pl.multiple_of(lb * BLK, BLK)
                pltpu.make_async_copy(
                    cent_hbm.at[:, pl.ds(start, BLK)], slab_v.at[s], sems[s]
                ).start()

            fire_rows(s, st, jnp.minimum(eb, WAVE))
            return st, eb

        def fire_rows(s, estart, n):
            def fire(e, _):
                pk = arena_v[pl.ds(estart + e, LANES)][0]
                pltpu.make_async_copy(
                    feat_hbm.at[pl.ds(
                        lax.shift_right_logical(pk, 7) * feat_dim, feat_dim)],
                    fr_v.at[s, pl.ds(e * feat_dim, feat_dim)],
                    sems[s],
                ).start()
                return 0

            lax.fori_loop(0, n, fire, 0)

        def drain_rows(s, n):
            def dr(e, _):
                pltpu.make_async_copy(
                    feat_hbm.at[pl.ds(0, feat_dim)],
                    fr_v.at[s, pl.ds(e * feat_dim, feat_dim)],
                    sems[s],
                ).wait()
                return 0

            lax.fori_loop(0, n, dr, 0)

        def process_wave(s, estart, n, carry):
            def body(e, carry):
                tot, acc = carry
                pk = arena_v[pl.ds(estart + e, LANES)][0]
                li = jnp.bitwise_and(pk, BLK - 1)
                li_v = jnp.full((LANES,), 0, jnp.int32) + li
                fbase = e * feat_dim
                sff_v = zeros
                sfc_v = zeros
                for c in range(nslices):
                    fv = fr_v[s, pl.ds(fbase + c * LANES, LANES)]
                    cv = plsc.load_gather(
                        slab_v.at[s], [frange + c * LANES, li_v])
                    sff_v = sff_v + fv * fv
                    sfc_v = sfc_v + fv * cv
                    acc = acc + cv * cv
                sff = jnp.sum(sff_v)
                sfc = jnp.sum(sfc_v)
                inv = _rsqrt_newton(jnp.maximum(sff, 1e-24))
                tot = tot + (sff * inv) * inv - (2.0 * inv) * sfc
                return tot, acc

            return lax.fori_loop(0, n, body, carry)

        def consume(s, start, eb, carry):
            @pl.when(eb > 0)
            def _():
                pltpu.make_async_copy(
                    cent_hbm.at[:, pl.ds(0, BLK)], slab_v.at[s], sems[s]
                ).wait()

            n0 = jnp.minimum(eb, WAVE)
            drain_rows(s, n0)
            carry = process_wave(s, start, n0, carry)

            def wcond(st):
                return st[0] < eb

            def wbody(st):
                ws, tot, acc = st
                n = jnp.minimum(eb - ws, WAVE)
                fire_rows(s, start + ws, n)
                drain_rows(s, n)
                tot, acc = process_wave(s, start + ws, n, (tot, acc))
                return ws + n, tot, acc

            _, tot, acc = lax.while_loop(
                wcond, wbody, (n0, carry[0], carry[1]))
            return tot, acc

        fzero = jnp.float32(0.0)
        sts = []
        for j in range(DEPTH):
            st, eb = stage(jnp.int32(j), j)
            sts.extend([st, eb])

        def step(m, carry):
            *se, tot, acc = carry
            se = list(se)
            for j in range(DEPTH):
                b = m * DEPTH + j
                tot, acc = consume(j, se[2 * j], se[2 * j + 1], (tot, acc))
                st, eb = stage(b + DEPTH, j)
                se[2 * j] = st
                se[2 * j + 1] = eb
            return (*se, tot, acc)

        out = lax.fori_loop(0, npb // DEPTH, step, (*sts, fzero, zeros))
        tot, acc = out[-2], out[-1]
        out_v[...] = acc + tot * (1.0 / LANES)
        pltpu.sync_copy(out_v, out_hbm.at[wid])

    return sc_kernel


def kernel(features, labels, centers):
    batch, feat_dim = features.shape
    num_classes = centers.shape[0]
    sc_call = _make_sc_call(batch, feat_dim, num_classes)
    partials = sc_call(
        features.reshape(-1), labels.astype(jnp.int32), centers.T
    )
    return jnp.sum(partials) / batch


# speedup vs baseline: 1.0522x; 1.0522x over previous
import functools

import jax
import jax.numpy as jnp
from jax import lax
from jax.experimental import pallas as pl
from jax.experimental.pallas import tpu as pltpu
from jax.experimental.pallas import tpu_sc as plsc

NUM_CORES = 2
NUM_SUBCORES = 16
LANES = 16
NUM_WORKERS = NUM_CORES * NUM_SUBCORES
BLK = 128
WAVE = 16
DEPTH = 4
NBKT = 16


def _rsqrt_newton(x):
    i = lax.bitcast_convert_type(x, jnp.int32)
    i = jnp.int32(0x5F3759DF) - lax.shift_right_logical(i, 1)
    y = lax.bitcast_convert_type(i, jnp.float32)
    for _ in range(3):
        y = y * (1.5 - 0.5 * x * y * y)
    return y


def _make_sc_call(batch, feat_dim, num_classes):
    nblocks = (num_classes + BLK - 1) // BLK
    npb = -(-((nblocks + NUM_WORKERS - 1) // NUM_WORKERS) // DEPTH) * DEPTH
    nslices = feat_dim // LANES

    mesh = plsc.VectorSubcoreMesh(core_axis_name="c", subcore_axis_name="s")

    @functools.partial(
        pl.kernel,
        mesh=mesh,
        compiler_params=pltpu.CompilerParams(needs_layout_passes=False),
        out_type=jax.ShapeDtypeStruct((NUM_WORKERS, LANES), jnp.float32),
        scratch_types=[
            pltpu.VMEM((batch + LANES,), jnp.int32),
            pltpu.VMEM((batch + LANES,), jnp.int32),
            pltpu.VMEM((batch + LANES,), jnp.int32),
            pltpu.VMEM((batch + LANES,), jnp.int32),
            pltpu.VMEM((batch + LANES,), jnp.int32),
            pltpu.VMEM((DEPTH, feat_dim, BLK), jnp.float32),
            pltpu.VMEM((DEPTH, WAVE * feat_dim), jnp.float32),
            pltpu.VMEM((LANES,), jnp.float32),
            pltpu.SMEM((256,), jnp.int32),
            pltpu.SemaphoreType.DMA,
            pltpu.SemaphoreType.DMA,
            pltpu.SemaphoreType.DMA,
            pltpu.SemaphoreType.DMA,
        ],
    )
    def sc_kernel(feat_hbm, lab_hbm, cent_hbm, out_hbm, lab_v, mylab_v,
                  myidx_v, bidx_v, arena_v, slab_v, fr_v, out_v, off_s,
                  *sems):
        wid = lax.axis_index("s") * NUM_CORES + lax.axis_index("c")
        pltpu.sync_copy(lab_hbm, lab_v.at[pl.ds(0, batch)])

        frange = lax.iota(jnp.int32, LANES)
        zeros = jnp.zeros((LANES,), jnp.float32)

        def popc(mask):
            return plsc.all_reduce_population_count(mask)[0]

        def scan_body(v, cnt):
            lv = lab_v[pl.ds(v * LANES, LANES)]
            m = jnp.bitwise_and(lax.shift_right_logical(lv, 7),
                                NUM_WORKERS - 1) == wid
            plsc.store_compressed(mylab_v.at[pl.ds(cnt, LANES)], lv, mask=m)
            plsc.store_compressed(myidx_v.at[pl.ds(cnt, LANES)],
                                  v * LANES + frange, mask=m)
            return cnt + popc(m)

        cnt = lax.fori_loop(0, batch // LANES, scan_body, jnp.int32(0))
        nv = lax.shift_right_logical(cnt + LANES - 1, 4)

        def bcount(v, cs):
            lv = mylab_v[pl.ds(v * LANES, LANES)]
            valid = (v * LANES + frange) < cnt
            t = lax.shift_right_logical(lv, 16)
            return tuple(
                cs[j] + popc(jnp.logical_and(t == j, valid))
                for j in range(NBKT)
            )

        bcnt = lax.fori_loop(0, nv, bcount, (jnp.int32(0),) * NBKT)
        boff = [jnp.int32(0)]
        for j in range(NBKT):
            boff.append(boff[j] + bcnt[j])

        def bplace(v, cur):
            lv = mylab_v[pl.ds(v * LANES, LANES)]
            iv = myidx_v[pl.ds(v * LANES, LANES)]
            valid = (v * LANES + frange) < cnt
            t = lax.shift_right_logical(lv, 16)
            out = []
            for j in range(NBKT):
                m = jnp.logical_and(t == j, valid)
                plsc.store_compressed(lab_v.at[pl.ds(cur[j], LANES)],
                                      lv, mask=m)
                plsc.store_compressed(bidx_v.at[pl.ds(cur[j], LANES)],
                                      iv, mask=m)
                out.append(cur[j] + popc(m))
            return tuple(out)

        lax.fori_loop(0, nv, bplace, tuple(boff[:NBKT]))

        for j in range(NBKT):
            lo = boff[j]
            hi = boff[j + 1]
            nvj = lax.shift_right_logical(hi - lo + LANES - 1, 4)

            def ccount(u, cs, lo=lo, hi=hi):
                p = lo + u * LANES
                lv = lab_v[pl.ds(p, LANES)]
                valid = (p + frange) < hi
                k = jnp.bitwise_and(lax.shift_right_logical(lv, 12),
                                    NBKT - 1)
                return tuple(
                    cs[q] + popc(jnp.logical_and(k == q, valid))
                    for q in range(NBKT)
                )

            ccnt = lax.fori_loop(0, nvj, ccount, (jnp.int32(0),) * NBKT)
            base = lo
            curs = []
            for q in range(NBKT):
                off_s[j * NBKT + q] = base
                curs.append(base)
                base = base + ccnt[q]

            def cplace(u, cur, lo=lo, hi=hi):
                p = lo + u * LANES
                lv = lab_v[pl.ds(p, LANES)]
                iv = bidx_v[pl.ds(p, LANES)]
                valid = (p + frange) < hi
                k = jnp.bitwise_and(lax.shift_right_logical(lv, 12),
                                    NBKT - 1)
                pk = iv * BLK + jnp.bitwise_and(lv, BLK - 1)
                out = []
                for q in range(NBKT):
                    m = jnp.logical_and(k == q, valid)
                    plsc.store_compressed(arena_v.at[pl.ds(cur[q], LANES)],
                                          pk, mask=m)
                    out.append(cur[q] + popc(m))
                return tuple(out)

            lax.fori_loop(0, nvj, cplace, tuple(curs))

        def stage(b_loc, s):
            lb = b_loc * NUM_WORKERS + wid
            st = off_s[b_loc]
            eb = off_s[b_loc + 1] - st

            @pl.when(eb > 0)
            def _():
                start = pl.multiple_of(lb * BLK, BLK)
                pltpu.make_async_copy(
                    cent_hbm.at[:, pl.ds(start, BLK)], slab_v.at[s], sems[s]
                ).start()

            fire_rows(s, st, jnp.minimum(eb, WAVE))
            return st, eb

        def fire_rows(s, estart, n):
            def fire(e, _):
                pk = arena_v[pl.ds(estart + e, LANES)][0]
                pltpu.make_async_copy(
                    feat_hbm.at[pl.ds(
                        lax.shift_right_logical(pk, 7) * feat_dim, feat_dim)],
                    fr_v.at[s, pl.ds(e * feat_dim, feat_dim)],
                    sems[s],
                ).start()
                return 0

            lax.fori_loop(0, n, fire, 0)

        def drain_rows(s, n):
            def dr(e, _):
                pltpu.make_async_copy(
                    feat_hbm.at[pl.ds(0, feat_dim)],
                    fr_v.at[s, pl.ds(e * feat_dim, feat_dim)],
                    sems[s],
                ).wait()
                return 0

            lax.fori_loop(0, n, dr, 0)

        def process_wave(s, estart, n, carry):
            def body(e, carry):
                tot, acc = carry
                pk = arena_v[pl.ds(estart + e, LANES)][0]
                li = jnp.bitwise_and(pk, BLK - 1)
                li_v = jnp.full((LANES,), 0, jnp.int32) + li
                fbase = e * feat_dim
                sff_v = zeros
                sfc_v = zeros
                for c in range(nslices):
                    fv = fr_v[s, pl.ds(fbase + c * LANES, LANES)]
                    cv = plsc.load_gather(
                        slab_v.at[s], [frange + c * LANES, li_v])
                    sff_v = sff_v + fv * fv
                    sfc_v = sfc_v + fv * cv
                    acc = acc + cv * cv
                sff = jnp.sum(sff_v)
                sfc = jnp.sum(sfc_v)
                inv = _rsqrt_newton(jnp.maximum(sff, 1e-24))
                tot = tot + (sff * inv) * inv - (2.0 * inv) * sfc
                return tot, acc

            return lax.fori_loop(0, n, body, carry)

        def consume(s, start, eb, carry):
            @pl.when(eb > 0)
            def _():
                pltpu.make_async_copy(
                    cent_hbm.at[:, pl.ds(0, BLK)], slab_v.at[s], sems[s]
                ).wait()

            n0 = jnp.minimum(eb, WAVE)
            drain_rows(s, n0)
            carry = process_wave(s, start, n0, carry)

            def wcond(st):
                return st[0] < eb

            def wbody(st):
                ws, tot, acc = st
                n = jnp.minimum(eb - ws, WAVE)
                fire_rows(s, start + ws, n)
                drain_rows(s, n)
                tot, acc = process_wave(s, start + ws, n, (tot, acc))
                return ws + n, tot, acc

            _, tot, acc = lax.while_loop(
                wcond, wbody, (n0, carry[0], carry[1]))
            return tot, acc

        fzero = jnp.float32(0.0)
        sts = []
        for j in range(DEPTH):
            st, eb = stage(jnp.int32(j), j)
            sts.extend([st, eb])

        def step(m, carry):
            *se, tot, acc = carry
            se = list(se)
            for j in range(DEPTH):
                b = m * DEPTH + j
                tot, acc = consume(j, se[2 * j], se[2 * j + 1], (tot, acc))
                st, eb = stage(b + DEPTH, j)
                se[2 * j] = st
                se[2 * j + 1] = eb
            return (*se, tot, acc)

        out = lax.fori_loop(0, npb // DEPTH, step, (*sts, fzero, zeros))
        tot, acc = out[-2], out[-1]
        out_v[...] = acc + tot * (1.0 / LANES)
        pltpu.sync_copy(out_v, out_hbm.at[wid])

    return sc_kernel


def kernel(features, labels, centers):
    batch, feat_dim = features.shape
    num_classes = centers.shape[0]
    sc_call = _make_sc_call(batch, feat_dim, num_classes)
    partials = sc_call(
        features.reshape(-1), labels.astype(jnp.int32), centers.T
    )
    return jnp.sum(partials) / batch
